# baseline (speedup 1.0000x reference)
import functools

import jax
import jax.numpy as jnp
from jax import lax
from jax.experimental import pallas as pl
from jax.experimental.pallas import tpu as pltpu
from jax.experimental.pallas import tpu_sc as plsc

_VOCAB = 100000
_MAXLEN = 200
_EMBED = 128
_BATCH = 1024

_NC, _NS = 2, 16
_NW = _NC * _NS
_ROWS = _BATCH * _MAXLEN
_CHUNK = 128
_NCHUNK = _ROWS // _CHUNK
_CPW = _NCHUNK // _NW
_LANES = 16
_DSL = _EMBED // _LANES
_NBUF = 3

_mesh = plsc.VectorSubcoreMesh(
    core_axis_name="c", subcore_axis_name="s",
    num_cores=_NC, num_subcores=_NS,
)


@functools.partial(
    pl.kernel,
    out_type=jax.ShapeDtypeStruct((_NCHUNK, _CHUNK, _EMBED), jnp.float32),
    mesh=_mesh,
    scratch_types=[
        pltpu.VMEM((2 * _MAXLEN, _EMBED), jnp.float32),
        pltpu.VMEM((_CPW, _CHUNK), jnp.int32),
        pltpu.VMEM((_CHUNK, _EMBED), jnp.float32),
        pltpu.VMEM((_CHUNK, _EMBED), jnp.float32),
        pltpu.VMEM((_CHUNK, _EMBED), jnp.float32),
        pltpu.SemaphoreType.DMA,
        pltpu.SemaphoreType.DMA,
        pltpu.SemaphoreType.DMA,
        pltpu.SemaphoreType.DMA,
        pltpu.SemaphoreType.DMA,
        pltpu.SemaphoreType.DMA,
    ],
)
def _embed_kernel(x_hbm, tok_hbm, pos_hbm, out_hbm,
                  pos2_v, idxs_v, buf0_v, buf1_v, buf2_v,
                  g0, g1, g2, w0, w1, w2):
    wid = lax.axis_index("s") * _NC + lax.axis_index("c")
    pltpu.sync_copy(x_hbm.at[wid], idxs_v)

    out_base = wid * _CPW
    bufs = (buf0_v, buf1_v, buf2_v)
    gsems = (g0, g1, g2)
    wsems = (w0, w1, w2)

    def start_gather(j, b):
        pltpu.async_copy(tok_hbm.at[idxs_v.at[j]], bufs[b], gsems[b])

    def wait_gather(j, b):
        pltpu.make_async_copy(tok_hbm.at[idxs_v.at[j]], bufs[b], gsems[b]).wait()

    def start_wb(j, b):
        pltpu.async_copy(bufs[b], out_hbm.at[out_base + j], wsems[b])

    def wait_wb(j, b):
        pltpu.make_async_copy(bufs[b], out_hbm.at[out_base + j], wsems[b]).wait()

    def add_pos(j, b):
        base_mod = lax.rem((out_base + j) * _CHUNK, _MAXLEN)

        @plsc.parallel_loop(0, _CHUNK)
        def _add(r):
            pr = base_mod + r
            for d in range(_DSL):
                sl = pl.ds(d * _LANES, _LANES)
                plsc.addupdate(bufs[b].at[r, sl], pos2_v[pr, sl])

    for b in range(_NBUF):
        start_gather(b, b)
    pltpu.sync_copy(pos_hbm, pos2_v.at[pl.ds(0, _MAXLEN)])
    pltpu.sync_copy(pos_hbm, pos2_v.at[pl.ds(_MAXLEN, _MAXLEN)])

    def triple_body(k, carry):
        for b in range(_NBUF):
            j = _NBUF * k + b

            @pl.when(j >= _NBUF - 1)
            def _prefetch():
                wait_wb(j - 2, (b + 1) % _NBUF)
                start_gather(j + 1, (b + 1) % _NBUF)

            wait_gather(j, b)
            add_pos(j, b)
            start_wb(j, b)
        return carry

    lax.fori_loop(0, (_CPW - 2) // _NBUF, triple_body, 0)

    wait_wb(46, 1)
    start_gather(49, 1)
    wait_gather(48, 0)
    add_pos(48, 0)
    start_wb(48, 0)
    wait_gather(49, 1)
    add_pos(49, 1)
    start_wb(49, 1)

    wait_wb(47, 2)
    wait_wb(48, 0)
    wait_wb(49, 1)


def kernel(x, token_table, pos_table):
    x3 = x.astype(jnp.int32).reshape(_NW, _CPW, _CHUNK)
    out = _embed_kernel(x3, token_table, pos_table)
    return out.reshape(_BATCH, _MAXLEN, _EMBED)


# speedup vs baseline: 1.0070x; 1.0070x over previous
import functools

import jax
import jax.numpy as jnp
from jax import lax
from jax.experimental import pallas as pl
from jax.experimental.pallas import tpu as pltpu
from jax.experimental.pallas import tpu_sc as plsc

_VOCAB = 100000
_MAXLEN = 200
_EMBED = 128
_BATCH = 1024

_NC, _NS = 2, 16
_NW = _NC * _NS
_ROWS = _BATCH * _MAXLEN
_CHUNK = 128
_NCHUNK = _ROWS // _CHUNK
_CPW = _NCHUNK // _NW
_LANES = 16
_DSL = _EMBED // _LANES
_NBUF = 3

_mesh = plsc.VectorSubcoreMesh(
    core_axis_name="c", subcore_axis_name="s",
    num_cores=_NC, num_subcores=_NS,
)


@functools.partial(
    pl.kernel,
    out_type=jax.ShapeDtypeStruct((_NCHUNK, _CHUNK, _EMBED), jnp.float32),
    mesh=_mesh,
    scratch_types=[
        pltpu.VMEM((2 * _MAXLEN, _EMBED), jnp.float32),
        pltpu.VMEM((_CPW, _CHUNK), jnp.int32),
        pltpu.VMEM((_CHUNK, _EMBED), jnp.float32),
        pltpu.VMEM((_CHUNK, _EMBED), jnp.float32),
        pltpu.VMEM((_CHUNK, _EMBED), jnp.float32),
        pltpu.SemaphoreType.DMA,
        pltpu.SemaphoreType.DMA,
        pltpu.SemaphoreType.DMA,
        pltpu.SemaphoreType.DMA,
        pltpu.SemaphoreType.DMA,
        pltpu.SemaphoreType.DMA,
    ],
)
def _embed_kernel(x_hbm, tok_hbm, pos_hbm, out_hbm,
                  pos2_v, idxs_v, buf0_v, buf1_v, buf2_v,
                  g0, g1, g2, w0, w1, w2):
    wid = lax.axis_index("s") * _NC + lax.axis_index("c")
    pltpu.sync_copy(pos_hbm, pos2_v.at[pl.ds(0, _MAXLEN)])
    pltpu.sync_copy(pos_hbm, pos2_v.at[pl.ds(_MAXLEN, _MAXLEN)])
    pltpu.sync_copy(x_hbm.at[wid], idxs_v)

    out_base = wid * _CPW
    bufs = (buf0_v, buf1_v, buf2_v)
    gsems = (g0, g1, g2)
    wsems = (w0, w1, w2)

    def start_gather(j, b):
        pltpu.async_copy(tok_hbm.at[idxs_v.at[j]], bufs[b], gsems[b])

    def wait_gather(j, b):
        pltpu.make_async_copy(tok_hbm.at[idxs_v.at[j]], bufs[b], gsems[b]).wait()

    def start_wb(j, b):
        pltpu.async_copy(bufs[b], out_hbm.at[out_base + j], wsems[b])

    def wait_wb(j, b):
        pltpu.make_async_copy(bufs[b], out_hbm.at[out_base + j], wsems[b]).wait()

    def add_pos(j, b):
        base_mod = lax.rem((out_base + j) * _CHUNK, _MAXLEN)

        @plsc.parallel_loop(0, _CHUNK)
        def _add(r):
            pr = base_mod + r
            for d in range(_DSL):
                sl = pl.ds(d * _LANES, _LANES)
                plsc.addupdate(bufs[b].at[r, sl], pos2_v[pr, sl])

    for b in range(_NBUF):
        start_gather(b, b)

    def triple_body(k, carry):
        for b in range(_NBUF):
            j = _NBUF * k + b

            @pl.when(j >= _NBUF - 1)
            def _prefetch():
                wait_wb(j - 2, (b + 1) % _NBUF)
                start_gather(j + 1, (b + 1) % _NBUF)

            wait_gather(j, b)
            add_pos(j, b)
            start_wb(j, b)
        return carry

    lax.fori_loop(0, (_CPW - 2) // _NBUF, triple_body, 0)

    wait_wb(46, 1)
    start_gather(49, 1)
    wait_gather(48, 0)
    add_pos(48, 0)
    start_wb(48, 0)
    wait_gather(49, 1)
    add_pos(49, 1)
    start_wb(49, 1)

    wait_wb(47, 2)
    wait_wb(48, 0)
    wait_wb(49, 1)


def kernel(x, token_table, pos_table):
    x3 = x.astype(jnp.int32).reshape(_NW, _CPW, _CHUNK)
    out = _embed_kernel(x3, token_table, pos_table)
    return out.reshape(_BATCH, _MAXLEN, _EMBED)


# speedup vs baseline: 1.0774x; 1.0699x over previous
import functools

import jax
import jax.numpy as jnp
from jax import lax
from jax.experimental import pallas as pl
from jax.experimental.pallas import tpu as pltpu
from jax.experimental.pallas import tpu_sc as plsc

_VOCAB = 100000
_MAXLEN = 200
_EMBED = 128
_BATCH = 1024

_NC, _NS = 2, 16
_NW = _NC * _NS
_ROWS = _BATCH * _MAXLEN
_CHUNK = 128
_NCHUNK = _ROWS // _CHUNK
_CPW = _NCHUNK // _NW
_LANES = 16
_DSL = _EMBED // _LANES
_NBUF = 4
_POS2 = _MAXLEN + _CHUNK

_mesh = plsc.VectorSubcoreMesh(
    core_axis_name="c", subcore_axis_name="s",
    num_cores=_NC, num_subcores=_NS,
)


@functools.partial(
    pl.kernel,
    out_type=jax.ShapeDtypeStruct((_NCHUNK, _CHUNK, _EMBED), jnp.float32),
    mesh=_mesh,
    scratch_types=[
        pltpu.VMEM((_POS2, _EMBED), jnp.float32),
        pltpu.VMEM((_CPW, _CHUNK), jnp.int32),
        pltpu.VMEM((_CHUNK, _EMBED), jnp.float32),
        pltpu.VMEM((_CHUNK, _EMBED), jnp.float32),
        pltpu.VMEM((_CHUNK, _EMBED), jnp.float32),
        pltpu.VMEM((_CHUNK, _EMBED), jnp.float32),
        pltpu.SemaphoreType.DMA,
        pltpu.SemaphoreType.DMA,
        pltpu.SemaphoreType.DMA,
        pltpu.SemaphoreType.DMA,
        pltpu.SemaphoreType.DMA,
        pltpu.SemaphoreType.DMA,
        pltpu.SemaphoreType.DMA,
        pltpu.SemaphoreType.DMA,
    ],
)
def _embed_kernel(x_hbm, tok_hbm, pos_hbm, out_hbm,
                  pos2_v, idxs_v, buf0_v, buf1_v, buf2_v, buf3_v,
                  g0, g1, g2, g3, w0, w1, w2, w3):
    wid = lax.axis_index("s") * _NC + lax.axis_index("c")
    pltpu.sync_copy(pos_hbm, pos2_v.at[pl.ds(0, _MAXLEN)])
    pltpu.sync_copy(pos_hbm.at[pl.ds(0, _CHUNK)],
                    pos2_v.at[pl.ds(_MAXLEN, _CHUNK)])
    pltpu.sync_copy(x_hbm.at[wid], idxs_v)

    out_base = wid * _CPW
    bufs = (buf0_v, buf1_v, buf2_v, buf3_v)
    gsems = (g0, g1, g2, g3)
    wsems = (w0, w1, w2, w3)

    def start_gather(j, b):
        pltpu.async_copy(tok_hbm.at[idxs_v.at[j]], bufs[b], gsems[b])

    def wait_gather(j, b):
        pltpu.make_async_copy(tok_hbm.at[idxs_v.at[j]], bufs[b], gsems[b]).wait()

    def start_wb(j, b):
        pltpu.async_copy(bufs[b], out_hbm.at[out_base + j], wsems[b])

    def wait_wb(j, b):
        pltpu.make_async_copy(bufs[b], out_hbm.at[out_base + j], wsems[b]).wait()

    def add_pos(j, b):
        base_mod = lax.rem((out_base + j) * _CHUNK, _MAXLEN)

        @plsc.parallel_loop(0, _CHUNK)
        def _add(r):
            pr = base_mod + r
            for d in range(_DSL):
                sl = pl.ds(d * _LANES, _LANES)
                plsc.addupdate(bufs[b].at[r, sl], pos2_v[pr, sl])

    start_gather(0, 0)
    start_gather(1, 1)

    def quad_body(k, carry):
        for b in range(_NBUF):
            j = _NBUF * k + b

            nb = (b + 2) % _NBUF

            @pl.when(j >= 2)
            def _drain():
                wait_wb(j - 2, nb)

            start_gather(j + 2, nb)

            wait_gather(j, b)
            add_pos(j, b)
            start_wb(j, b)
        return carry

    lax.fori_loop(0, (_CPW - 2) // _NBUF, quad_body, 0)

    wait_gather(48, 0)
    add_pos(48, 0)
    start_wb(48, 0)
    wait_gather(49, 1)
    add_pos(49, 1)
    start_wb(49, 1)

    wait_wb(46, 2)
    wait_wb(47, 3)
    wait_wb(48, 0)
    wait_wb(49, 1)


def kernel(x, token_table, pos_table):
    x3 = x.astype(jnp.int32).reshape(_NW, _CPW, _CHUNK)
    out = _embed_kernel(x3, token_table, pos_table)
    return out.reshape(_BATCH, _MAXLEN, _EMBED)


# speedup vs baseline: 1.0981x; 1.0192x over previous
import functools

import jax
import jax.numpy as jnp
from jax import lax
from jax.experimental import pallas as pl
from jax.experimental.pallas import tpu as pltpu
from jax.experimental.pallas import tpu_sc as plsc

_VOCAB = 100000
_MAXLEN = 200
_EMBED = 128
_BATCH = 1024

_NC, _NS = 2, 16
_NW = _NC * _NS
_ROWS = _BATCH * _MAXLEN
_CHUNK = 128
_NCHUNK = _ROWS // _CHUNK
_CPW = _NCHUNK // _NW
_LANES = 16
_DSL = _EMBED // _LANES
_NBUF = 5
_DIST = 3

_mesh = plsc.VectorSubcoreMesh(
    core_axis_name="c", subcore_axis_name="s",
    num_cores=_NC, num_subcores=_NS,
)


@functools.partial(
    pl.kernel,
    out_type=jax.ShapeDtypeStruct((_NCHUNK, _CHUNK, _EMBED), jnp.float32),
    mesh=_mesh,
    scratch_types=[
        pltpu.VMEM((_MAXLEN, _EMBED), jnp.float32),
        pltpu.VMEM((_CPW, _CHUNK), jnp.int32),
        pltpu.VMEM((_CHUNK, _EMBED), jnp.float32),
        pltpu.VMEM((_CHUNK, _EMBED), jnp.float32),
        pltpu.VMEM((_CHUNK, _EMBED), jnp.float32),
        pltpu.VMEM((_CHUNK, _EMBED), jnp.float32),
        pltpu.VMEM((_CHUNK, _EMBED), jnp.float32),
        pltpu.SemaphoreType.DMA,
        pltpu.SemaphoreType.DMA,
        pltpu.SemaphoreType.DMA,
        pltpu.SemaphoreType.DMA,
        pltpu.SemaphoreType.DMA,
        pltpu.SemaphoreType.DMA,
        pltpu.SemaphoreType.DMA,
        pltpu.SemaphoreType.DMA,
        pltpu.SemaphoreType.DMA,
        pltpu.SemaphoreType.DMA,
    ],
)
def _embed_kernel(x_hbm, tok_hbm, pos_hbm, out_hbm,
                  pos_v, idxs_v, buf0_v, buf1_v, buf2_v, buf3_v, buf4_v,
                  g0, g1, g2, g3, g4, w0, w1, w2, w3, w4):
    wid = lax.axis_index("s") * _NC + lax.axis_index("c")
    pltpu.sync_copy(pos_hbm, pos_v)
    pltpu.sync_copy(x_hbm.at[wid], idxs_v)

    out_base = wid * _CPW
    bufs = (buf0_v, buf1_v, buf2_v, buf3_v, buf4_v)
    gsems = (g0, g1, g2, g3, g4)
    wsems = (w0, w1, w2, w3, w4)

    def start_gather(j, b):
        pltpu.async_copy(tok_hbm.at[idxs_v.at[j]], bufs[b], gsems[b])

    def wait_gather(j, b):
        pltpu.make_async_copy(tok_hbm.at[idxs_v.at[j]], bufs[b], gsems[b]).wait()

    def start_wb(j, b):
        pltpu.async_copy(bufs[b], out_hbm.at[out_base + j], wsems[b])

    def wait_wb(j, b):
        pltpu.make_async_copy(bufs[b], out_hbm.at[out_base + j], wsems[b]).wait()

    def add_pos(j, b):
        base_mod = lax.rem((out_base + j) * _CHUNK, _MAXLEN)

        @plsc.parallel_loop(0, _CHUNK)
        def _add(r):
            pr0 = base_mod + r
            pr = pr0 - jnp.where(pr0 >= _MAXLEN, _MAXLEN, 0)
            for d in range(_DSL):
                sl = pl.ds(d * _LANES, _LANES)
                plsc.addupdate(bufs[b].at[r, sl], pos_v[pr, sl])

    for b in range(_DIST):
        start_gather(b, b)

    def body(k, carry):
        for b in range(_NBUF):
            j = _NBUF * k + b

            nb = (b + _DIST) % _NBUF

            @pl.when(j >= 2)
            def _drain():
                wait_wb(j - 2, nb)

            start_gather(j + _DIST, nb)

            wait_gather(j, b)
            add_pos(j, b)
            start_wb(j, b)
        return carry

    lax.fori_loop(0, (_CPW - _NBUF) // _NBUF, body, 0)

    wait_wb(43, 3)
    start_gather(48, 3)
    wait_gather(45, 0)
    add_pos(45, 0)
    start_wb(45, 0)

    wait_wb(44, 4)
    start_gather(49, 4)
    wait_gather(46, 1)
    add_pos(46, 1)
    start_wb(46, 1)

    for j, b in ((47, 2), (48, 3), (49, 4)):
        wait_gather(j, b)
        add_pos(j, b)
        start_wb(j, b)

    for j, b in ((45, 0), (46, 1), (47, 2), (48, 3), (49, 4)):
        wait_wb(j, b)


def kernel(x, token_table, pos_table):
    x3 = x.astype(jnp.int32).reshape(_NW, _CPW, _CHUNK)
    out = _embed_kernel(x3, token_table, pos_table)
    return out.reshape(_BATCH, _MAXLEN, _EMBED)
